# baseline (speedup 1.0000x reference)
import functools

import jax
import jax.numpy as jnp
from jax import lax
from jax.experimental import pallas as pl
from jax.experimental.pallas import tpu as pltpu
from jax.experimental.pallas import tpu_sc as plsc

_N = 10000
_DIM = 64
_TOPK = 16
_ROW_BLK = 400
_BN_EPS = 1e-5
_NW = 32
_NCHUNK = 2 * _N // 16


def _norms_body(w_ref, n_ref):
    w = w_ref[...]
    n_ref[0, :] = jnp.sqrt(jnp.sum(w * w, axis=1))


def _col_norms(emb):
    return pl.pallas_call(
        _norms_body,
        out_shape=jax.ShapeDtypeStruct((1, _N), jnp.float32),
    )(emb)


def _topk_body(w_blk_ref, w_full_ref, nf_ref, idx_ref):
    wb = w_blk_ref[...]
    wf = w_full_ref[...]
    dots = lax.dot_general(wb, wf, (((1,), (1,)), ((), ())),
                           preferred_element_type=jnp.float32)
    nb = jnp.sqrt(jnp.sum(wb * wb, axis=1))
    nf = nf_ref[0, :]
    cos = dots / (nb[:, None] * nf[None, :])
    col = lax.broadcasted_iota(jnp.int32, cos.shape, 1)
    idxs = []
    for _ in range(_TOPK):
        idx = jnp.argmax(cos, axis=1)[:, None]
        idxs.append(idx)
        cos = jnp.where(col == idx, jnp.float32(-3.0), cos)
    idxs.append(jnp.zeros((wb.shape[0], 128 - _TOPK), jnp.int32))
    idx_ref[...] = jnp.concatenate(idxs, axis=1)


def _learn_topk(emb):
    grid = _N // _ROW_BLK
    nf = _col_norms(emb)
    return pl.pallas_call(
        _topk_body,
        grid=(grid,),
        in_specs=[
            pl.BlockSpec((_ROW_BLK, _DIM), lambda i: (i, 0)),
            pl.BlockSpec((_N, _DIM), lambda i: (0, 0)),
            pl.BlockSpec((1, _N), lambda i: (0, 0)),
        ],
        out_specs=pl.BlockSpec((_ROW_BLK, 128), lambda i: (i, 0)),
        out_shape=jax.ShapeDtypeStruct((_N, 128), jnp.int32),
    )(emb, emb, nf)[:, :_TOPK]


def _proj_body(x_ref, emb_ref, wlin_ref, attm_ref, xl_ref, adst_ref, asrc_ref):
    x = x_ref[0]
    wlin = wlin_ref[...]
    xl = jnp.dot(x, wlin, preferred_element_type=jnp.float32)
    xl_ref[0] = jnp.concatenate([xl, jnp.zeros_like(xl)], axis=1)
    cat = jnp.concatenate([xl, emb_ref[...]], axis=1)
    a2 = jnp.dot(cat, attm_ref[...], preferred_element_type=jnp.float32)
    adst_ref[0, 0, :] = a2[:, 0]
    asrc_ref[0, 0, :] = a2[:, 1]


def _project(data, emb, W_lin, att_i, att_j, att_em_i, att_em_j):
    B = data.shape[0]
    cat_i = jnp.concatenate([att_i, att_em_i])
    cat_j = jnp.concatenate([att_j, att_em_j])
    attm = jnp.zeros((2 * _DIM, 128), jnp.float32)
    attm = attm.at[:, 0].set(cat_i).at[:, 1].set(cat_j)
    return pl.pallas_call(
        _proj_body,
        grid=(B,),
        in_specs=[
            pl.BlockSpec((1, _N, 128), lambda b: (b, 0, 0)),
            pl.BlockSpec((_N, _DIM), lambda b: (0, 0)),
            pl.BlockSpec((128, _DIM), lambda b: (0, 0)),
            pl.BlockSpec((2 * _DIM, 128), lambda b: (0, 0)),
        ],
        out_specs=[
            pl.BlockSpec((1, _N, 128), lambda b: (b, 0, 0)),
            pl.BlockSpec((1, 1, _N), lambda b: (b, 0, 0)),
            pl.BlockSpec((1, 1, _N), lambda b: (b, 0, 0)),
        ],
        out_shape=[
            jax.ShapeDtypeStruct((B, _N, 128), jnp.float32),
            jax.ShapeDtypeStruct((B, 1, _N), jnp.float32),
            jax.ShapeDtypeStruct((B, 1, _N), jnp.float32),
        ],
    )(data, emb, W_lin, attm)


def _mp_body(eidx_hbm, xl_hbm, asrc_hbm, adst_hbm, emb_hbm, cvec_hbm, out_hbm,
             asrc_v, adst_v, cvec_v, idxf_v, rows_v, embc_v, att_v, out_v, sem):
    cid = lax.axis_index("c")
    sid = lax.axis_index("s")
    wid = sid * 2 + cid
    _KIDX = [jnp.full((16,), k, jnp.int32) for k in range(_TOPK)]
    pltpu.sync_copy(asrc_hbm, asrc_v)
    pltpu.sync_copy(adst_hbm, adst_v)
    pltpu.sync_copy(cvec_hbm, cvec_v)
    nloops = (_NCHUNK + _NW - 1) // _NW

    def chunk_body(ci, carry):
        c = wid + ci * _NW

        @pl.when(c < _NCHUNK)
        def _():
            nb = c * 16
            ebase = nb - jnp.where(nb >= _N, _N, 0)
            pltpu.sync_copy(eidx_hbm.at[pl.ds(c * 256, 256)], idxf_v)
            g1 = pltpu.async_copy(xl_hbm.at[idxf_v.at[pl.ds(0, 128)]],
                                  rows_v.at[pl.ds(0, 128)], sem)
            g2 = pltpu.async_copy(xl_hbm.at[idxf_v.at[pl.ds(128, 128)]],
                                  rows_v.at[pl.ds(128, 128)], sem)
            pltpu.sync_copy(emb_hbm.at[pl.ds(ebase, 16)], embc_v)
            g1.wait()
            g2.wait()

            lane = lax.iota(jnp.int32, 16)
            outacc = jnp.zeros((16,), jnp.float32)
            for n in range(16):
                idxn = idxf_v[pl.ds(n * 16, 16)]
                a_vals = plsc.load_gather(asrc_v, [idxn])
                a_d = plsc.load_gather(adst_v,
                                       [jnp.full((16,), nb + n, jnp.int32)])
                alpha = a_d + a_vals
                alpha = jnp.where(alpha >= 0, alpha, alpha * jnp.float32(0.2))
                m = jnp.max(alpha)
                ex = jnp.exp(alpha - m)
                den = jnp.sum(ex)
                att = ex / (den + jnp.float32(1e-16))
                accs = [jnp.zeros((16,), jnp.float32) for _ in range(4)]
                for k in range(_TOPK):
                    wk = att[k]
                    for cs in range(4):
                        accs[cs] = accs[cs] + wk * rows_v[n * 16 + k, pl.ds(cs * 16, 16)]
                yv = jnp.zeros((16,), jnp.float32)
                for cs in range(4):
                    sl = pl.ds(cs * 16, 16)
                    t = (accs[cs] + cvec_v[0, sl]) * cvec_v[1, sl] + cvec_v[2, sl]
                    t = jnp.maximum(t, jnp.float32(0.0))
                    t = t * embc_v[n, sl]
                    t = t * cvec_v[3, sl] + cvec_v[4, sl]
                    t = jnp.maximum(t, jnp.float32(0.0))
                    yv = yv + t * cvec_v[5, sl]
                yv = yv + cvec_v[6, pl.ds(0, 16)]
                y = jnp.sum(yv)
                outacc = jnp.where(lane == n, y, outacc)
            out_v[...] = outacc
            pltpu.sync_copy(out_v, out_hbm.at[pl.ds(nb, 16)])

        return carry

    lax.fori_loop(0, nloops, chunk_body, 0)


def _message_pass(eidx, xl_flat, asrc, adst, emb, cvec):
    mesh = plsc.VectorSubcoreMesh(core_axis_name="c", subcore_axis_name="s")
    f = functools.partial(
        pl.kernel,
        out_type=jax.ShapeDtypeStruct((2 * _N,), jnp.float32),
        mesh=mesh,
        compiler_params=pltpu.CompilerParams(needs_layout_passes=False),
        scratch_types=[
            pltpu.VMEM((2 * _N,), jnp.float32),
            pltpu.VMEM((2 * _N,), jnp.float32),
            pltpu.VMEM((7, _DIM), jnp.float32),
            pltpu.VMEM((256,), jnp.int32),
            pltpu.VMEM((256, 128), jnp.float32),
            pltpu.VMEM((16, _DIM), jnp.float32),
            pltpu.VMEM((16,), jnp.float32),
            pltpu.VMEM((16,), jnp.float32),
            pltpu.SemaphoreType.DMA,
        ],
    )(_mp_body)
    return f(eidx, xl_flat, asrc, adst, emb, cvec)


def kernel(data, org_edge_index, embedding, W_lin, att_i, att_j, att_em_i,
           att_em_j, gnn_bias, bn_w, bn_b, bno_w, bno_b, W_out, b_out):
    B, N, F = data.shape
    topk_idx = _learn_topk(embedding)
    xl, adst, asrc = _project(data, embedding, W_lin, att_i, att_j,
                              att_em_i, att_em_j)

    eidx = jnp.concatenate([topk_idx, topk_idx + _N], axis=0).reshape(-1)
    s1 = bn_w / jnp.sqrt(1.0 + _BN_EPS)
    s2 = bno_w / jnp.sqrt(1.0 + _BN_EPS)
    cvec = jnp.stack([gnn_bias, s1, bn_b, s2, bno_b, W_out[:, 0],
                      jnp.broadcast_to(b_out / 16.0, (_DIM,))], axis=0)

    out = _message_pass(eidx, xl.reshape(-1, 128), asrc.reshape(-1),
                        adst.reshape(-1), embedding, cvec)
    return out.reshape(B, N)


# speedup vs baseline: 1.0733x; 1.0733x over previous
import functools

import jax
import jax.numpy as jnp
from jax import lax
from jax.experimental import pallas as pl
from jax.experimental.pallas import tpu as pltpu
from jax.experimental.pallas import tpu_sc as plsc

_N = 10000
_DIM = 64
_TOPK = 16
_ROW_BLK = 80
_BN_EPS = 1e-5
_NW = 32
_NCHUNK = 2 * _N // 16


def _norms_body(w_ref, n_ref):
    w = w_ref[...]
    n_ref[0, :] = jnp.sqrt(jnp.sum(w * w, axis=1))


def _col_norms(emb):
    return pl.pallas_call(
        _norms_body,
        out_shape=jax.ShapeDtypeStruct((1, _N), jnp.float32),
    )(emb)


def _topk_body(w_blk_ref, w_full_ref, nf_ref, idx_ref):
    wb = w_blk_ref[...]
    wf = w_full_ref[...]
    dots = lax.dot_general(wb, wf, (((1,), (1,)), ((), ())),
                           preferred_element_type=jnp.float32)
    nb = jnp.sqrt(jnp.sum(wb * wb, axis=1))
    nf = nf_ref[0, :]
    cos = dots / (nb[:, None] * nf[None, :])
    col = lax.broadcasted_iota(jnp.int32, cos.shape, 1)
    idxs = []
    for _ in range(_TOPK):
        idx = jnp.argmax(cos, axis=1)[:, None]
        idxs.append(idx)
        cos = jnp.where(col == idx, jnp.float32(-3.0), cos)
    idxs.append(jnp.zeros((wb.shape[0], 128 - _TOPK), jnp.int32))
    idx_ref[...] = jnp.concatenate(idxs, axis=1)


def _learn_topk(emb):
    grid = _N // _ROW_BLK
    nf = _col_norms(emb)
    return pl.pallas_call(
        _topk_body,
        grid=(grid,),
        in_specs=[
            pl.BlockSpec((_ROW_BLK, _DIM), lambda i: (i, 0)),
            pl.BlockSpec((_N, _DIM), lambda i: (0, 0)),
            pl.BlockSpec((1, _N), lambda i: (0, 0)),
        ],
        out_specs=pl.BlockSpec((_ROW_BLK, 128), lambda i: (i, 0)),
        out_shape=jax.ShapeDtypeStruct((_N, 128), jnp.int32),
    )(emb, emb, nf)[:, :_TOPK]


def _proj_body(x_ref, emb_ref, wlin_ref, attm_ref, xl_ref, adst_ref, asrc_ref):
    x = x_ref[0]
    wlin = wlin_ref[...]
    xl = jnp.dot(x, wlin, preferred_element_type=jnp.float32)
    xl_ref[0] = jnp.concatenate([xl, jnp.zeros_like(xl)], axis=1)
    cat = jnp.concatenate([xl, emb_ref[...]], axis=1)
    a2 = jnp.dot(cat, attm_ref[...], preferred_element_type=jnp.float32)
    adst_ref[0, 0, :] = a2[:, 0]
    asrc_ref[0, 0, :] = a2[:, 1]


def _project(data, emb, W_lin, att_i, att_j, att_em_i, att_em_j):
    B = data.shape[0]
    cat_i = jnp.concatenate([att_i, att_em_i])
    cat_j = jnp.concatenate([att_j, att_em_j])
    attm = jnp.zeros((2 * _DIM, 128), jnp.float32)
    attm = attm.at[:, 0].set(cat_i).at[:, 1].set(cat_j)
    return pl.pallas_call(
        _proj_body,
        grid=(B,),
        in_specs=[
            pl.BlockSpec((1, _N, 128), lambda b: (b, 0, 0)),
            pl.BlockSpec((_N, _DIM), lambda b: (0, 0)),
            pl.BlockSpec((128, _DIM), lambda b: (0, 0)),
            pl.BlockSpec((2 * _DIM, 128), lambda b: (0, 0)),
        ],
        out_specs=[
            pl.BlockSpec((1, _N, 128), lambda b: (b, 0, 0)),
            pl.BlockSpec((1, 1, _N), lambda b: (b, 0, 0)),
            pl.BlockSpec((1, 1, _N), lambda b: (b, 0, 0)),
        ],
        out_shape=[
            jax.ShapeDtypeStruct((B, _N, 128), jnp.float32),
            jax.ShapeDtypeStruct((B, 1, _N), jnp.float32),
            jax.ShapeDtypeStruct((B, 1, _N), jnp.float32),
        ],
    )(data, emb, W_lin, attm)


def _mp_body(eidx_hbm, xl_hbm, asrc_hbm, adst_hbm, emb_hbm, cvec_hbm, out_hbm,
             asrc_v, adst_v, cvec_v, idxf_v, rows_v, embc_v, att_v, out_v, sem):
    cid = lax.axis_index("c")
    sid = lax.axis_index("s")
    wid = sid * 2 + cid
    _KIDX = [jnp.full((16,), k, jnp.int32) for k in range(_TOPK)]
    pltpu.sync_copy(asrc_hbm, asrc_v)
    pltpu.sync_copy(adst_hbm, adst_v)
    pltpu.sync_copy(cvec_hbm, cvec_v)
    nloops = (_NCHUNK + _NW - 1) // _NW

    def chunk_body(ci, carry):
        c = wid + ci * _NW

        @pl.when(c < _NCHUNK)
        def _():
            nb = c * 16
            ebase = nb - jnp.where(nb >= _N, _N, 0)
            pltpu.sync_copy(eidx_hbm.at[pl.ds(c * 256, 256)], idxf_v)
            g1 = pltpu.async_copy(xl_hbm.at[idxf_v.at[pl.ds(0, 128)]],
                                  rows_v.at[pl.ds(0, 128)], sem)
            g2 = pltpu.async_copy(xl_hbm.at[idxf_v.at[pl.ds(128, 128)]],
                                  rows_v.at[pl.ds(128, 128)], sem)
            pltpu.sync_copy(emb_hbm.at[pl.ds(ebase, 16)], embc_v)
            g1.wait()
            g2.wait()

            lane = lax.iota(jnp.int32, 16)
            outacc = jnp.zeros((16,), jnp.float32)
            for n in range(16):
                idxn = idxf_v[pl.ds(n * 16, 16)]
                a_vals = plsc.load_gather(asrc_v, [idxn])
                a_d = plsc.load_gather(adst_v,
                                       [jnp.full((16,), nb + n, jnp.int32)])
                alpha = a_d + a_vals
                alpha = jnp.where(alpha >= 0, alpha, alpha * jnp.float32(0.2))
                m = jnp.max(alpha)
                ex = jnp.exp(alpha - m)
                den = jnp.sum(ex)
                att = ex / (den + jnp.float32(1e-16))
                accs = [jnp.zeros((16,), jnp.float32) for _ in range(4)]
                for k in range(_TOPK):
                    wk = att[k]
                    for cs in range(4):
                        accs[cs] = accs[cs] + wk * rows_v[n * 16 + k, pl.ds(cs * 16, 16)]
                yv = jnp.zeros((16,), jnp.float32)
                for cs in range(4):
                    sl = pl.ds(cs * 16, 16)
                    t = (accs[cs] + cvec_v[0, sl]) * cvec_v[1, sl] + cvec_v[2, sl]
                    t = jnp.maximum(t, jnp.float32(0.0))
                    t = t * embc_v[n, sl]
                    t = t * cvec_v[3, sl] + cvec_v[4, sl]
                    t = jnp.maximum(t, jnp.float32(0.0))
                    yv = yv + t * cvec_v[5, sl]
                yv = yv + cvec_v[6, pl.ds(0, 16)]
                y = jnp.sum(yv)
                outacc = jnp.where(lane == n, y, outacc)
            out_v[...] = outacc
            pltpu.sync_copy(out_v, out_hbm.at[pl.ds(nb, 16)])

        return carry

    lax.fori_loop(0, nloops, chunk_body, 0)


def _message_pass(eidx, xl_flat, asrc, adst, emb, cvec):
    mesh = plsc.VectorSubcoreMesh(core_axis_name="c", subcore_axis_name="s")
    f = functools.partial(
        pl.kernel,
        out_type=jax.ShapeDtypeStruct((2 * _N,), jnp.float32),
        mesh=mesh,
        compiler_params=pltpu.CompilerParams(needs_layout_passes=False),
        scratch_types=[
            pltpu.VMEM((2 * _N,), jnp.float32),
            pltpu.VMEM((2 * _N,), jnp.float32),
            pltpu.VMEM((7, _DIM), jnp.float32),
            pltpu.VMEM((256,), jnp.int32),
            pltpu.VMEM((256, 128), jnp.float32),
            pltpu.VMEM((16, _DIM), jnp.float32),
            pltpu.VMEM((16,), jnp.float32),
            pltpu.VMEM((16,), jnp.float32),
            pltpu.SemaphoreType.DMA,
        ],
    )(_mp_body)
    return f(eidx, xl_flat, asrc, adst, emb, cvec)


def kernel(data, org_edge_index, embedding, W_lin, att_i, att_j, att_em_i,
           att_em_j, gnn_bias, bn_w, bn_b, bno_w, bno_b, W_out, b_out):
    B, N, F = data.shape
    topk_idx = _learn_topk(embedding)
    xl, adst, asrc = _project(data, embedding, W_lin, att_i, att_j,
                              att_em_i, att_em_j)

    eidx = jnp.concatenate([topk_idx, topk_idx + _N], axis=0).reshape(-1)
    s1 = bn_w / jnp.sqrt(1.0 + _BN_EPS)
    s2 = bno_w / jnp.sqrt(1.0 + _BN_EPS)
    cvec = jnp.stack([gnn_bias, s1, bn_b, s2, bno_b, W_out[:, 0],
                      jnp.broadcast_to(b_out / 16.0, (_DIM,))], axis=0)

    out = _message_pass(eidx, xl.reshape(-1, 128), asrc.reshape(-1),
                        adst.reshape(-1), embedding, cvec)
    return out.reshape(B, N)


# speedup vs baseline: 1.1175x; 1.0412x over previous
import functools

import jax
import jax.numpy as jnp
from jax import lax
from jax.experimental import pallas as pl
from jax.experimental.pallas import tpu as pltpu
from jax.experimental.pallas import tpu_sc as plsc

_N = 10000
_DIM = 64
_TOPK = 16
_ROW_BLK = 200
_BN_EPS = 1e-5
_NW = 32
_NCHUNK = 2 * _N // 16


def _norms_body(w_ref, n_ref):
    w = w_ref[...]
    n_ref[0, :] = jnp.sqrt(jnp.sum(w * w, axis=1))


def _col_norms(emb):
    return pl.pallas_call(
        _norms_body,
        out_shape=jax.ShapeDtypeStruct((1, _N), jnp.float32),
    )(emb)


def _topk_body(w_blk_ref, w_full_ref, nf_ref, idx_ref):
    wb = w_blk_ref[...]
    wf = w_full_ref[...]
    dots = lax.dot_general(wb, wf, (((1,), (1,)), ((), ())),
                           preferred_element_type=jnp.float32)
    nb = jnp.sqrt(jnp.sum(wb * wb, axis=1))
    nf = nf_ref[0, :]
    cos = dots / (nb[:, None] * nf[None, :])
    col = lax.broadcasted_iota(jnp.int32, cos.shape, 1)
    idxs = []
    for _ in range(_TOPK):
        idx = jnp.argmax(cos, axis=1)[:, None]
        idxs.append(idx)
        cos = jnp.where(col == idx, jnp.float32(-3.0), cos)
    idxs.append(jnp.zeros((wb.shape[0], 128 - _TOPK), jnp.int32))
    idx_ref[...] = jnp.concatenate(idxs, axis=1)


def _learn_topk(emb):
    grid = _N // _ROW_BLK
    nf = _col_norms(emb)
    return pl.pallas_call(
        _topk_body,
        grid=(grid,),
        in_specs=[
            pl.BlockSpec((_ROW_BLK, _DIM), lambda i: (i, 0)),
            pl.BlockSpec((_N, _DIM), lambda i: (0, 0)),
            pl.BlockSpec((1, _N), lambda i: (0, 0)),
        ],
        out_specs=pl.BlockSpec((_ROW_BLK, 128), lambda i: (i, 0)),
        out_shape=jax.ShapeDtypeStruct((_N, 128), jnp.int32),
    )(emb, emb, nf)[:, :_TOPK]


def _proj_body(x_ref, emb_ref, wlin_ref, attm_ref, xl_ref, adst_ref, asrc_ref):
    x = x_ref[0]
    wlin = wlin_ref[...]
    xl = jnp.dot(x, wlin, preferred_element_type=jnp.float32)
    xl_ref[0] = jnp.concatenate([xl, jnp.zeros_like(xl)], axis=1)
    cat = jnp.concatenate([xl, emb_ref[...]], axis=1)
    a2 = jnp.dot(cat, attm_ref[...], preferred_element_type=jnp.float32)
    adst_ref[0, 0, :] = a2[:, 0]
    asrc_ref[0, 0, :] = a2[:, 1]


def _project(data, emb, W_lin, att_i, att_j, att_em_i, att_em_j):
    B = data.shape[0]
    cat_i = jnp.concatenate([att_i, att_em_i])
    cat_j = jnp.concatenate([att_j, att_em_j])
    attm = jnp.zeros((2 * _DIM, 128), jnp.float32)
    attm = attm.at[:, 0].set(cat_i).at[:, 1].set(cat_j)
    return pl.pallas_call(
        _proj_body,
        grid=(B,),
        in_specs=[
            pl.BlockSpec((1, _N, 128), lambda b: (b, 0, 0)),
            pl.BlockSpec((_N, _DIM), lambda b: (0, 0)),
            pl.BlockSpec((128, _DIM), lambda b: (0, 0)),
            pl.BlockSpec((2 * _DIM, 128), lambda b: (0, 0)),
        ],
        out_specs=[
            pl.BlockSpec((1, _N, 128), lambda b: (b, 0, 0)),
            pl.BlockSpec((1, 1, _N), lambda b: (b, 0, 0)),
            pl.BlockSpec((1, 1, _N), lambda b: (b, 0, 0)),
        ],
        out_shape=[
            jax.ShapeDtypeStruct((B, _N, 128), jnp.float32),
            jax.ShapeDtypeStruct((B, 1, _N), jnp.float32),
            jax.ShapeDtypeStruct((B, 1, _N), jnp.float32),
        ],
    )(data, emb, W_lin, attm)


def _mp_body(eidx_hbm, xl_hbm, asrc_hbm, adst_hbm, emb_hbm, cvec_hbm, out_hbm,
             asrc_v, adst_v, cvec_v, idxf_v, rows_v, embc_v, out_v, sem):
    cid = lax.axis_index("c")
    sid = lax.axis_index("s")
    wid = sid * 2 + cid
    pltpu.sync_copy(asrc_hbm, asrc_v)
    pltpu.sync_copy(adst_hbm, adst_v)
    pltpu.sync_copy(cvec_hbm, cvec_v)
    nloops = (_NCHUNK + _NW - 1) // _NW

    def chunk_body(ci, carry):
        c = wid + ci * _NW

        @pl.when(c < _NCHUNK)
        def _():
            nb = c * 16
            ebase = nb - jnp.where(nb >= _N, _N, 0)
            pltpu.sync_copy(eidx_hbm.at[pl.ds(c * 256, 256)], idxf_v)
            g1 = pltpu.async_copy(xl_hbm.at[idxf_v.at[pl.ds(0, 128)]],
                                  rows_v.at[pl.ds(0, 128)], sem)
            g2 = pltpu.async_copy(xl_hbm.at[idxf_v.at[pl.ds(128, 128)]],
                                  rows_v.at[pl.ds(128, 128)], sem)
            pltpu.sync_copy(emb_hbm.at[pl.ds(ebase, 16)], embc_v)
            g1.wait()
            g2.wait()

            lane = lax.iota(jnp.int32, 16)
            outacc = jnp.zeros((16,), jnp.float32)
            for n in range(16):
                idxn = idxf_v[pl.ds(n * 16, 16)]
                a_vals = plsc.load_gather(asrc_v, [idxn])
                a_d = plsc.load_gather(adst_v,
                                       [jnp.full((16,), nb + n, jnp.int32)])
                alpha = a_d + a_vals
                alpha = jnp.where(alpha >= 0, alpha, alpha * jnp.float32(0.2))
                m = jnp.max(alpha)
                ex = jnp.exp(alpha - m)
                den = jnp.sum(ex)
                att = ex / (den + jnp.float32(1e-16))
                accs = [jnp.zeros((16,), jnp.float32) for _ in range(4)]
                for k in range(_TOPK):
                    wk = att[k]
                    for cs in range(4):
                        accs[cs] = accs[cs] + wk * rows_v[n * 16 + k, pl.ds(cs * 16, 16)]
                yv = jnp.zeros((16,), jnp.float32)
                for cs in range(4):
                    sl = pl.ds(cs * 16, 16)
                    t = (accs[cs] + cvec_v[0, sl]) * cvec_v[1, sl] + cvec_v[2, sl]
                    t = jnp.maximum(t, jnp.float32(0.0))
                    t = t * embc_v[n, sl]
                    t = t * cvec_v[3, sl] + cvec_v[4, sl]
                    t = jnp.maximum(t, jnp.float32(0.0))
                    yv = yv + t * cvec_v[5, sl]
                yv = yv + cvec_v[6, pl.ds(0, 16)]
                y = jnp.sum(yv)
                outacc = jnp.where(lane == n, y, outacc)
            out_v[...] = outacc
            pltpu.sync_copy(out_v, out_hbm.at[pl.ds(nb, 16)])

        return carry

    lax.fori_loop(0, nloops, chunk_body, 0)


def _message_pass(eidx, xl_flat, asrc, adst, emb, cvec):
    mesh = plsc.VectorSubcoreMesh(core_axis_name="c", subcore_axis_name="s")
    f = functools.partial(
        pl.kernel,
        out_type=jax.ShapeDtypeStruct((2 * _N,), jnp.float32),
        mesh=mesh,
        compiler_params=pltpu.CompilerParams(needs_layout_passes=False),
        scratch_types=[
            pltpu.VMEM((2 * _N,), jnp.float32),
            pltpu.VMEM((2 * _N,), jnp.float32),
            pltpu.VMEM((7, _DIM), jnp.float32),
            pltpu.VMEM((256,), jnp.int32),
            pltpu.VMEM((256, 128), jnp.float32),
            pltpu.VMEM((16, _DIM), jnp.float32),
            pltpu.VMEM((16,), jnp.float32),
            pltpu.SemaphoreType.DMA,
        ],
    )(_mp_body)
    return f(eidx, xl_flat, asrc, adst, emb, cvec)


def kernel(data, org_edge_index, embedding, W_lin, att_i, att_j, att_em_i,
           att_em_j, gnn_bias, bn_w, bn_b, bno_w, bno_b, W_out, b_out):
    B, N, F = data.shape
    topk_idx = _learn_topk(embedding)
    xl, adst, asrc = _project(data, embedding, W_lin, att_i, att_j,
                              att_em_i, att_em_j)

    eidx = jnp.concatenate([topk_idx, topk_idx + _N], axis=0).reshape(-1)
    s1 = bn_w / jnp.sqrt(1.0 + _BN_EPS)
    s2 = bno_w / jnp.sqrt(1.0 + _BN_EPS)
    cvec = jnp.stack([gnn_bias, s1, bn_b, s2, bno_b, W_out[:, 0],
                      jnp.broadcast_to(b_out / 16.0, (_DIM,))], axis=0)

    out = _message_pass(eidx, xl.reshape(-1, 128), asrc.reshape(-1),
                        adst.reshape(-1), embedding, cvec)
    return out.reshape(B, N)
